# baseline (speedup 1.0000x reference)
import functools
from functools import partial

import jax
import jax.numpy as jnp
from jax import lax
from jax.experimental import pallas as pl
from jax.experimental.pallas import tpu as pltpu
from jax.experimental.pallas import tpu_sc as plsc

N = 10000
E = 320000
DIM = 128
AGG = 64
CLS = 32
ROUNDS = 20

NC = 2
NS = 16
NW = NC * NS
CH = 128
CPW = 2 * (-(-E // (NW * CH * 2)))
NGRP = CPW // 2
CPAD = CPW + 2
EPAD = NW * CPAD * CH
NACC = 10112
RPS = NACC // NS

RBLK = 2000


def _msg_body(pre_hbm, dst_hbm, src_hbm, zeros_hbm, out_hbm,
              acc, dst_v, src_v, rows_v, sems):
    cid = lax.axis_index("c")
    sid = lax.axis_index("s")
    wid = cid * NS + sid

    pltpu.sync_copy(zeros_hbm.at[pl.ds(sid * RPS, RPS)],
                    acc.at[pl.ds(sid * RPS, RPS)])

    dsts, srcs, rows, dsem, ssem, gsem = (
        dst_v[0], dst_v[1]), (src_v[0], src_v[1]), rows_v, sems[0:2], sems[2:4], sems[4:6]

    def idx_start(b, j):
        pltpu.async_copy(dst_hbm.at[wid, j], dsts[b], dsem[b])
        pltpu.async_copy(src_hbm.at[wid, j], srcs[b], ssem[b])

    def idx_wait(b, j):
        pltpu.make_async_copy(dst_hbm.at[wid, j], dsts[b], dsem[b]).wait()
        pltpu.make_async_copy(src_hbm.at[wid, j], srcs[b], ssem[b]).wait()

    def gather_start(b):
        pltpu.async_copy(pre_hbm.at[srcs[b]], rows[b], gsem[b])

    def gather_wait(b):
        pltpu.make_async_copy(pre_hbm.at[srcs[b]], rows[b], gsem[b]).wait()

    def scatter(b):
        pltpu.sync_copy(rows[b], acc.at[dsts[b]], add=True)

    idx_start(0, 0)
    idx_start(1, 1)
    idx_wait(0, 0)
    gather_start(0)

    def pair(g, carry):
        for b in (0, 1):
            j = 2 * g + b
            nb = 1 - b
            idx_wait(nb, j + 1)
            gather_wait(b)
            gather_start(nb)
            scatter(b)
            idx_start(b, j + 2)
        return carry

    lax.fori_loop(0, NGRP, pair, 0)
    gather_wait(0)
    idx_wait(1, CPW + 1)
    plsc.subcore_barrier()

    pltpu.sync_copy(acc.at[pl.ds(sid * RPS, RPS)],
                    out_hbm.at[cid, pl.ds(sid * RPS, RPS)])


_msg_kernel = pl.kernel(
    _msg_body,
    out_type=jax.ShapeDtypeStruct((NC, NACC, DIM), jnp.float32),
    mesh=plsc.VectorSubcoreMesh(core_axis_name="c", subcore_axis_name="s"),
    scratch_types=[
        pltpu.VMEM_SHARED((NACC, DIM), jnp.float32),
        [pltpu.VMEM((CH,), jnp.int32) for _ in range(2)],
        [pltpu.VMEM((CH,), jnp.int32) for _ in range(2)],
        [pltpu.VMEM((CH, DIM), jnp.float32) for _ in range(2)],
        [pltpu.SemaphoreType.DMA for _ in range(6)],
    ],
)


def _init_body(feats, WiT, bi, W1T, b1, W2T, b2, h_out, pre_out):
    h = jnp.dot(feats[...], WiT[...], preferred_element_type=jnp.float32) + bi[...]
    h_out[...] = h
    a = jax.nn.relu(jnp.dot(h, W1T[...], preferred_element_type=jnp.float32) + b1[...])
    pre_out[...] = jnp.dot(a, W2T[...], preferred_element_type=jnp.float32) + b2[...]


def _fused_body(parts, h_ref, WgiT, WghT, bgi, bgh, W1T, b1, W2T, b2,
                h_out, pre_out):
    x = parts[0] + parts[1]
    h = h_ref[...]
    gi = jnp.dot(x, WgiT[...], preferred_element_type=jnp.float32) + bgi[...]
    gh = jnp.dot(h, WghT[...], preferred_element_type=jnp.float32) + bgh[...]
    r = jax.nn.sigmoid(gi[:, :DIM] + gh[:, :DIM])
    z = jax.nn.sigmoid(gi[:, DIM:2 * DIM] + gh[:, DIM:2 * DIM])
    n = jnp.tanh(gi[:, 2 * DIM:] + r * gh[:, 2 * DIM:])
    hn = (1.0 - z) * n + z * h
    h_out[...] = hn
    a = jax.nn.relu(jnp.dot(hn, W1T[...], preferred_element_type=jnp.float32) + b1[...])
    pre_out[...] = jnp.dot(a, W2T[...], preferred_element_type=jnp.float32) + b2[...]


def _cls_body(h_ref, W1T, b1, W2T, b2, out_ref):
    a = jax.nn.relu(jnp.dot(h_ref[...], W1T[...], preferred_element_type=jnp.float32) + b1[...])
    out_ref[...] = jnp.dot(a, W2T[...], preferred_element_type=jnp.float32) + b2[...]


def _row_spec(d):
    return pl.BlockSpec((RBLK, d), lambda i: (i, 0))


def _full_spec(shape):
    nd = len(shape)
    return pl.BlockSpec(shape, lambda i: (0,) * nd)


def _w(shape):
    return _full_spec(shape)


_GRID = (N // RBLK,)


def _init_call(feats, WiT, bi, W1T, b1, W2T, b2):
    return pl.pallas_call(
        _init_body,
        grid=_GRID,
        in_specs=[_row_spec(4), _w((4, DIM)), _w((1, DIM)),
                  _w((DIM, AGG)), _w((1, AGG)), _w((AGG, DIM)), _w((1, DIM))],
        out_specs=[_row_spec(DIM), _row_spec(DIM)],
        out_shape=[jax.ShapeDtypeStruct((N, DIM), jnp.float32),
                   jax.ShapeDtypeStruct((N, DIM), jnp.float32)],
    )(feats, WiT, bi, W1T, b1, W2T, b2)


def _fused_call(parts, h, WgiT, WghT, bgi, bgh, W1T, b1, W2T, b2):
    return pl.pallas_call(
        _fused_body,
        grid=_GRID,
        in_specs=[pl.BlockSpec((NC, RBLK, DIM), lambda i: (0, i, 0)),

                  _row_spec(DIM),
                  _w((DIM, 3 * DIM)), _w((DIM, 3 * DIM)),
                  _w((1, 3 * DIM)), _w((1, 3 * DIM)),
                  _w((DIM, AGG)), _w((1, AGG)), _w((AGG, DIM)), _w((1, DIM))],
        out_specs=[_row_spec(DIM), _row_spec(DIM)],
        out_shape=[jax.ShapeDtypeStruct((N, DIM), jnp.float32),
                   jax.ShapeDtypeStruct((N, DIM), jnp.float32)],
    )(parts, h, WgiT, WghT, bgi, bgh, W1T, b1, W2T, b2)


def _cls_call(h, W1T, b1, W2T, b2):
    return pl.pallas_call(
        _cls_body,
        grid=_GRID,
        in_specs=[_row_spec(DIM), _w((DIM, CLS)), _w((1, CLS)),
                  _w((CLS, 1)), _w((1, 1))],
        out_specs=[_row_spec(1)],
        out_shape=[jax.ShapeDtypeStruct((N, 1), jnp.float32)],
    )(h, W1T, b1, W2T, b2)[0]


def _pad_idx(idx, fill):
    main = jnp.concatenate(
        [idx, jnp.full((NW * CPW * CH - E,), fill, jnp.int32)]
    ).reshape(NW, CPW, CH)
    dummy = jnp.full((NW, CPAD - CPW, CH), fill, jnp.int32)
    return jnp.concatenate([main, dummy], axis=1)


def kernel(features, edge_index, W_init, b_init, Wf1, bf1, Wf2, bf2,
           Wb1, bb1, Wb2, bb2, Wfg_ih, Wfg_hh, bfg_ih, bfg_hh,
           Wbg_ih, Wbg_hh, bbg_ih, bbg_hh, Wc1, bc1, Wc2, bc2):
    row = edge_index[0]
    col = edge_index[1]
    f_dst = _pad_idx(row, N)
    f_src = _pad_idx(col, 0)
    b_dst = _pad_idx(col, N)
    b_src = _pad_idx(row, 0)
    zeros_tbl = jnp.zeros((NACC, DIM), jnp.float32)

    r2 = lambda b: b.reshape(1, -1)
    Wf1T, Wf2T = Wf1.T, Wf2.T
    Wb1T, Wb2T = Wb1.T, Wb2.T
    fg = (Wfg_ih.T, Wfg_hh.T, r2(bfg_ih), r2(bfg_hh))
    bg = (Wbg_ih.T, Wbg_hh.T, r2(bbg_ih), r2(bbg_hh))

    h, f_pre = _init_call(features, W_init.T, r2(b_init),
                          Wf1T, r2(bf1), Wf2T, r2(bf2))

    def round_body(_, carry):
        h, f_pre = carry
        f_parts = _msg_kernel(f_pre, f_dst, f_src, zeros_tbl)
        h, b_pre = _fused_call(f_parts, h, *fg, Wb1T, r2(bb1), Wb2T, r2(bb2))
        b_parts = _msg_kernel(b_pre, b_dst, b_src, zeros_tbl)
        h, f_pre = _fused_call(b_parts, h, *bg, Wf1T, r2(bf1), Wf2T, r2(bf2))
        return h, f_pre

    h, _ = lax.fori_loop(0, ROUNDS, round_body, (h, f_pre))
    return _cls_call(h, Wc1.T, r2(bc1), Wc2.T, r2(bc2))


# speedup vs baseline: 1.0982x; 1.0982x over previous
import functools
from functools import partial

import jax
import jax.numpy as jnp
from jax import lax
from jax.experimental import pallas as pl
from jax.experimental.pallas import tpu as pltpu
from jax.experimental.pallas import tpu_sc as plsc

N = 10000
E = 320000
DIM = 128
AGG = 64
CLS = 32
ROUNDS = 20

NC = 2
NS = 16
NW = NC * NS
CH = 128
CPW = 2 * (-(-E // (NW * CH * 2)))
NGRP = CPW // 2
CPAD = CPW + 2
EPAD = NW * CPAD * CH
NACC = 10112
RPS = NACC // NS

RBLK = 2000


def _msg_body(pre_hbm, dst_hbm, src_hbm, zeros_hbm, out_hbm,
              acc, dst_v, src_v, rows_v, sems):
    cid = lax.axis_index("c")
    sid = lax.axis_index("s")
    wid = cid * NS + sid

    pltpu.sync_copy(zeros_hbm.at[pl.ds(sid * RPS, RPS)],
                    acc.at[pl.ds(sid * RPS, RPS)])

    def chunk(j, carry):
        pltpu.sync_copy(dst_hbm.at[wid, j], dst_v[0])
        pltpu.sync_copy(src_hbm.at[wid, j], src_v[0])
        pltpu.async_copy(pre_hbm.at[src_v[0]], rows_v[0], sems[0]).wait()
        pltpu.sync_copy(rows_v[0], acc.at[dst_v[0]], add=True)
        return carry

    lax.fori_loop(0, CPW, chunk, 0)
    plsc.subcore_barrier()

    pltpu.sync_copy(acc.at[pl.ds(sid * RPS, RPS)],
                    out_hbm.at[cid, pl.ds(sid * RPS, RPS)])


_msg_kernel = pl.kernel(
    _msg_body,
    out_type=jax.ShapeDtypeStruct((NC, NACC, DIM), jnp.float32),
    mesh=plsc.VectorSubcoreMesh(core_axis_name="c", subcore_axis_name="s"),
    scratch_types=[
        pltpu.VMEM_SHARED((NACC, DIM), jnp.float32),
        [pltpu.VMEM((CH,), jnp.int32) for _ in range(2)],
        [pltpu.VMEM((CH,), jnp.int32) for _ in range(2)],
        [pltpu.VMEM((CH, DIM), jnp.float32) for _ in range(2)],
        [pltpu.SemaphoreType.DMA for _ in range(6)],
    ],
)


def _init_body(feats, WiT, bi, W1T, b1, W2T, b2, h_out, pre_out):
    h = jnp.dot(feats[...], WiT[...], preferred_element_type=jnp.float32) + bi[...]
    h_out[...] = h
    a = jax.nn.relu(jnp.dot(h, W1T[...], preferred_element_type=jnp.float32) + b1[...])
    pre_out[...] = jnp.dot(a, W2T[...], preferred_element_type=jnp.float32) + b2[...]


def _fused_body(parts, h_ref, WgiT, WghT, bgi, bgh, W1T, b1, W2T, b2,
                h_out, pre_out):
    x = parts[0] + parts[1]
    h = h_ref[...]
    gi = jnp.dot(x, WgiT[...], preferred_element_type=jnp.float32) + bgi[...]
    gh = jnp.dot(h, WghT[...], preferred_element_type=jnp.float32) + bgh[...]
    r = jax.nn.sigmoid(gi[:, :DIM] + gh[:, :DIM])
    z = jax.nn.sigmoid(gi[:, DIM:2 * DIM] + gh[:, DIM:2 * DIM])
    n = jnp.tanh(gi[:, 2 * DIM:] + r * gh[:, 2 * DIM:])
    hn = (1.0 - z) * n + z * h
    h_out[...] = hn
    a = jax.nn.relu(jnp.dot(hn, W1T[...], preferred_element_type=jnp.float32) + b1[...])
    pre_out[...] = jnp.dot(a, W2T[...], preferred_element_type=jnp.float32) + b2[...]


def _cls_body(h_ref, W1T, b1, W2T, b2, out_ref):
    a = jax.nn.relu(jnp.dot(h_ref[...], W1T[...], preferred_element_type=jnp.float32) + b1[...])
    out_ref[...] = jnp.dot(a, W2T[...], preferred_element_type=jnp.float32) + b2[...]


def _row_spec(d):
    return pl.BlockSpec((RBLK, d), lambda i: (i, 0))


def _full_spec(shape):
    nd = len(shape)
    return pl.BlockSpec(shape, lambda i: (0,) * nd)


def _w(shape):
    return _full_spec(shape)


_GRID = (N // RBLK,)


def _init_call(feats, WiT, bi, W1T, b1, W2T, b2):
    return pl.pallas_call(
        _init_body,
        grid=_GRID,
        in_specs=[_row_spec(4), _w((4, DIM)), _w((1, DIM)),
                  _w((DIM, AGG)), _w((1, AGG)), _w((AGG, DIM)), _w((1, DIM))],
        out_specs=[_row_spec(DIM), _row_spec(DIM)],
        out_shape=[jax.ShapeDtypeStruct((N, DIM), jnp.float32),
                   jax.ShapeDtypeStruct((N, DIM), jnp.float32)],
    )(feats, WiT, bi, W1T, b1, W2T, b2)


def _fused_call(parts, h, WgiT, WghT, bgi, bgh, W1T, b1, W2T, b2):
    return pl.pallas_call(
        _fused_body,
        grid=_GRID,
        in_specs=[pl.BlockSpec((NC, RBLK, DIM), lambda i: (0, i, 0)),

                  _row_spec(DIM),
                  _w((DIM, 3 * DIM)), _w((DIM, 3 * DIM)),
                  _w((1, 3 * DIM)), _w((1, 3 * DIM)),
                  _w((DIM, AGG)), _w((1, AGG)), _w((AGG, DIM)), _w((1, DIM))],
        out_specs=[_row_spec(DIM), _row_spec(DIM)],
        out_shape=[jax.ShapeDtypeStruct((N, DIM), jnp.float32),
                   jax.ShapeDtypeStruct((N, DIM), jnp.float32)],
    )(parts, h, WgiT, WghT, bgi, bgh, W1T, b1, W2T, b2)


def _cls_call(h, W1T, b1, W2T, b2):
    return pl.pallas_call(
        _cls_body,
        grid=_GRID,
        in_specs=[_row_spec(DIM), _w((DIM, CLS)), _w((1, CLS)),
                  _w((CLS, 1)), _w((1, 1))],
        out_specs=[_row_spec(1)],
        out_shape=[jax.ShapeDtypeStruct((N, 1), jnp.float32)],
    )(h, W1T, b1, W2T, b2)[0]


def _pad_idx(idx, fill):
    main = jnp.concatenate(
        [idx, jnp.full((NW * CPW * CH - E,), fill, jnp.int32)]
    ).reshape(NW, CPW, CH)
    dummy = jnp.full((NW, CPAD - CPW, CH), fill, jnp.int32)
    return jnp.concatenate([main, dummy], axis=1)


def kernel(features, edge_index, W_init, b_init, Wf1, bf1, Wf2, bf2,
           Wb1, bb1, Wb2, bb2, Wfg_ih, Wfg_hh, bfg_ih, bfg_hh,
           Wbg_ih, Wbg_hh, bbg_ih, bbg_hh, Wc1, bc1, Wc2, bc2):
    row = edge_index[0]
    col = edge_index[1]
    f_dst = _pad_idx(row, N)
    f_src = _pad_idx(col, 0)
    b_dst = _pad_idx(col, N)
    b_src = _pad_idx(row, 0)
    zeros_tbl = jnp.zeros((NACC, DIM), jnp.float32)

    r2 = lambda b: b.reshape(1, -1)
    Wf1T, Wf2T = Wf1.T, Wf2.T
    Wb1T, Wb2T = Wb1.T, Wb2.T
    fg = (Wfg_ih.T, Wfg_hh.T, r2(bfg_ih), r2(bfg_hh))
    bg = (Wbg_ih.T, Wbg_hh.T, r2(bbg_ih), r2(bbg_hh))

    h, f_pre = _init_call(features, W_init.T, r2(b_init),
                          Wf1T, r2(bf1), Wf2T, r2(bf2))

    def round_body(_, carry):
        h, f_pre = carry
        f_parts = _msg_kernel(f_pre, f_dst, f_src, zeros_tbl)
        h, b_pre = _fused_call(f_parts, h, *fg, Wb1T, r2(bb1), Wb2T, r2(bb2))
        b_parts = _msg_kernel(b_pre, b_dst, b_src, zeros_tbl)
        h, f_pre = _fused_call(b_parts, h, *bg, Wf1T, r2(bf1), Wf2T, r2(bf2))
        return h, f_pre

    h, _ = lax.fori_loop(0, ROUNDS, round_body, (h, f_pre))
    return _cls_call(h, Wc1.T, r2(bc1), Wc2.T, r2(bc2))


# speedup vs baseline: 2.3010x; 2.0952x over previous
import functools
from functools import partial

import jax
import jax.numpy as jnp
from jax import lax
from jax.experimental import pallas as pl
from jax.experimental.pallas import tpu as pltpu
from jax.experimental.pallas import tpu_sc as plsc

N = 10000
E = 320000
DIM = 128
AGG = 64
CLS = 32
ROUNDS = 20

NC = 2
NS = 16
NW = NC * NS
CH = 128
CPW = 2 * (-(-E // (NW * CH * 2)))
NGRP = CPW // 2
CPAD = CPW + 2
EPAD = NW * CPAD * CH
NACC = 10112
RPS = NACC // NS

RBLK = 2000


def _msg_body(pre_hbm, dst_hbm, src_hbm, zeros_hbm, out_hbm,
              acc, dst_v, src_v, rows_v, sems):
    cid = lax.axis_index("c")
    sid = lax.axis_index("s")
    wid = cid * NS + sid

    pltpu.sync_copy(zeros_hbm.at[pl.ds(sid * RPS, RPS)],
                    acc.at[pl.ds(sid * RPS, RPS)])

    def chunk(j, carry):
        pltpu.sync_copy(dst_hbm.at[wid, j], dst_v[0])
        pltpu.sync_copy(src_hbm.at[wid, j], src_v[0])
        pltpu.async_copy(pre_hbm.at[src_v[0]], rows_v[0], sems[0]).wait()
        pltpu.sync_copy(rows_v[0], acc.at[dst_v[0]], add=True)
        return carry

    lax.fori_loop(0, CPW, chunk, 0)
    plsc.subcore_barrier()

    pltpu.sync_copy(acc.at[pl.ds(sid * RPS, RPS)],
                    out_hbm.at[cid, pl.ds(sid * RPS, RPS)])


_msg_kernel = pl.kernel(
    _msg_body,
    out_type=jax.ShapeDtypeStruct((NC, NACC, DIM), jnp.float32),
    mesh=plsc.VectorSubcoreMesh(core_axis_name="c", subcore_axis_name="s"),
    scratch_types=[
        pltpu.VMEM_SHARED((NACC, DIM), jnp.float32),
        [pltpu.VMEM((CH,), jnp.int32) for _ in range(2)],
        [pltpu.VMEM((CH,), jnp.int32) for _ in range(2)],
        [pltpu.VMEM((CH, DIM), jnp.float32) for _ in range(2)],
        [pltpu.SemaphoreType.DMA for _ in range(6)],
    ],
)


def _init_body(feats, WiT, bi, W1T, b1, W2T, b2, h_out, pre_out):
    h = jnp.dot(feats[...], WiT[...], preferred_element_type=jnp.float32) + bi[...]
    h_out[...] = h
    a = jax.nn.relu(jnp.dot(h, W1T[...], preferred_element_type=jnp.float32) + b1[...])
    pre_out[...] = jnp.dot(a, W2T[...], preferred_element_type=jnp.float32) + b2[...]


def _fused_body(parts, h_ref, WgiT, WghT, bgi, bgh, W1T, b1, W2T, b2,
                h_out, pre_out):
    x = parts[0] + parts[1]
    h = h_ref[...]
    gi = jnp.dot(x, WgiT[...], preferred_element_type=jnp.float32) + bgi[...]
    gh = jnp.dot(h, WghT[...], preferred_element_type=jnp.float32) + bgh[...]
    r = jax.nn.sigmoid(gi[:, :DIM] + gh[:, :DIM])
    z = jax.nn.sigmoid(gi[:, DIM:2 * DIM] + gh[:, DIM:2 * DIM])
    n = jnp.tanh(gi[:, 2 * DIM:] + r * gh[:, 2 * DIM:])
    hn = (1.0 - z) * n + z * h
    h_out[...] = hn
    a = jax.nn.relu(jnp.dot(hn, W1T[...], preferred_element_type=jnp.float32) + b1[...])
    pre_out[...] = jnp.dot(a, W2T[...], preferred_element_type=jnp.float32) + b2[...]


def _cls_body(h_ref, W1T, b1, W2T, b2, out_ref):
    a = jax.nn.relu(jnp.dot(h_ref[...], W1T[...], preferred_element_type=jnp.float32) + b1[...])
    out_ref[...] = jnp.dot(a, W2T[...], preferred_element_type=jnp.float32) + b2[...]


def _row_spec(d):
    return pl.BlockSpec((RBLK, d), lambda i: (i, 0))


def _full_spec(shape):
    nd = len(shape)
    return pl.BlockSpec(shape, lambda i: (0,) * nd)


def _w(shape):
    return _full_spec(shape)


_GRID = (N // RBLK,)


def _init_call(feats, WiT, bi, W1T, b1, W2T, b2):
    return pl.pallas_call(
        _init_body,
        grid=_GRID,
        in_specs=[_row_spec(4), _w((4, DIM)), _w((1, DIM)),
                  _w((DIM, AGG)), _w((1, AGG)), _w((AGG, DIM)), _w((1, DIM))],
        out_specs=[_row_spec(DIM), _row_spec(DIM)],
        out_shape=[jax.ShapeDtypeStruct((N, DIM), jnp.float32),
                   jax.ShapeDtypeStruct((N, DIM), jnp.float32)],
    )(feats, WiT, bi, W1T, b1, W2T, b2)


def _fused_call(parts, h, WgiT, WghT, bgi, bgh, W1T, b1, W2T, b2):
    return pl.pallas_call(
        _fused_body,
        grid=_GRID,
        in_specs=[pl.BlockSpec((NC, RBLK, DIM), lambda i: (0, i, 0)),

                  _row_spec(DIM),
                  _w((DIM, 3 * DIM)), _w((DIM, 3 * DIM)),
                  _w((1, 3 * DIM)), _w((1, 3 * DIM)),
                  _w((DIM, AGG)), _w((1, AGG)), _w((AGG, DIM)), _w((1, DIM))],
        out_specs=[_row_spec(DIM), _row_spec(DIM)],
        out_shape=[jax.ShapeDtypeStruct((N, DIM), jnp.float32),
                   jax.ShapeDtypeStruct((N, DIM), jnp.float32)],
    )(parts, h, WgiT, WghT, bgi, bgh, W1T, b1, W2T, b2)


def _cls_call(h, W1T, b1, W2T, b2):
    return pl.pallas_call(
        _cls_body,
        grid=_GRID,
        in_specs=[_row_spec(DIM), _w((DIM, CLS)), _w((1, CLS)),
                  _w((CLS, 1)), _w((1, 1))],
        out_specs=[_row_spec(1)],
        out_shape=[jax.ShapeDtypeStruct((N, 1), jnp.float32)],
    )(h, W1T, b1, W2T, b2)[0]


EPW = E // NW
PPW = CPAD * CH - EPW


def _pad_idx(idx, dummy_vals):
    pad = jnp.broadcast_to(dummy_vals[None, :], (NW, PPW))
    return jnp.concatenate([idx.reshape(NW, EPW), pad], axis=1
                           ).reshape(NW, CPAD, CH)


_DST_PAD = (N + (jnp.arange(PPW) % (NACC - N))).astype(jnp.int32)
_SRC_PAD = (jnp.arange(PPW) % N).astype(jnp.int32)


def kernel(features, edge_index, W_init, b_init, Wf1, bf1, Wf2, bf2,
           Wb1, bb1, Wb2, bb2, Wfg_ih, Wfg_hh, bfg_ih, bfg_hh,
           Wbg_ih, Wbg_hh, bbg_ih, bbg_hh, Wc1, bc1, Wc2, bc2):
    row = edge_index[0]
    col = edge_index[1]
    f_dst = _pad_idx(row, _DST_PAD)
    f_src = _pad_idx(col, _SRC_PAD)
    b_dst = _pad_idx(col, _DST_PAD)
    b_src = _pad_idx(row, _SRC_PAD)
    zeros_tbl = jnp.zeros((NACC, DIM), jnp.float32)

    r2 = lambda b: b.reshape(1, -1)
    Wf1T, Wf2T = Wf1.T, Wf2.T
    Wb1T, Wb2T = Wb1.T, Wb2.T
    fg = (Wfg_ih.T, Wfg_hh.T, r2(bfg_ih), r2(bfg_hh))
    bg = (Wbg_ih.T, Wbg_hh.T, r2(bbg_ih), r2(bbg_hh))

    h, f_pre = _init_call(features, W_init.T, r2(b_init),
                          Wf1T, r2(bf1), Wf2T, r2(bf2))

    def round_body(_, carry):
        h, f_pre = carry
        f_parts = _msg_kernel(f_pre, f_dst, f_src, zeros_tbl)
        h, b_pre = _fused_call(f_parts, h, *fg, Wb1T, r2(bb1), Wb2T, r2(bb2))
        b_parts = _msg_kernel(b_pre, b_dst, b_src, zeros_tbl)
        h, f_pre = _fused_call(b_parts, h, *bg, Wf1T, r2(bf1), Wf2T, r2(bf2))
        return h, f_pre

    h, _ = lax.fori_loop(0, ROUNDS, round_body, (h, f_pre))
    return _cls_call(h, Wc1.T, r2(bc1), Wc2.T, r2(bc2))
